# Initial kernel scaffold; baseline (speedup 1.0000x reference)
import jax
import jax.numpy as jnp
from jax.experimental import pallas as pl


def kernel(x_s, x_t, edge_index, edge_attr, u, batch_s, batch_t, W1, b1, W2, b2):
    raise NotImplementedError("write your pallas kernel here")


# speedup vs baseline: 4.9696x; 4.9696x over previous
import functools

import jax
import jax.numpy as jnp
from jax import lax
from jax.experimental import pallas as pl
from jax.experimental.pallas import tpu as pltpu
from jax.experimental.pallas import tpu_sc as plsc

N = 100000
F = 128
B = 512
TILES = 16
CHUNK = 128
NCH = 49
PER_TILE = NCH * CHUNK
LAST_ROWS = N - 15 * PER_TILE
LAST_FULL = LAST_ROWS // CHUNK
LAST_REM = LAST_ROWS - LAST_FULL * CHUNK
ACC_ROWS = 528
ZROWS = B // TILES
CNT_W = 128


def _prep_ids(batch):
    ids = batch.astype(jnp.int32)
    pad = jnp.full((TILES * PER_TILE - N,), B, jnp.int32)
    return jnp.concatenate([ids, pad]).reshape(TILES, NCH, CHUNK)


def _sc_segment_sums(x_s, ids_s, x_t, ids_t, zeros_f, zeros_c, ones_c):
    mesh = plsc.VectorSubcoreMesh(core_axis_name="c", subcore_axis_name="s")

    @functools.partial(
        pl.kernel,
        out_type=(
            jax.ShapeDtypeStruct((B, F), jnp.float32),
            jax.ShapeDtypeStruct((B, CNT_W), jnp.float32),
            jax.ShapeDtypeStruct((B, F), jnp.float32),
            jax.ShapeDtypeStruct((B, CNT_W), jnp.float32),
        ),
        mesh=mesh,
        scratch_types=[
            pltpu.VMEM((NCH, CHUNK), jnp.int32),
            pltpu.VMEM((CHUNK, F), jnp.float32),
            pltpu.VMEM((CHUNK, CNT_W), jnp.float32),
            pltpu.VMEM_SHARED((ACC_ROWS, F), jnp.float32),
            pltpu.VMEM_SHARED((ACC_ROWS, CNT_W), jnp.float32),
        ],
    )
    def k(x_s_hbm, ids_s_hbm, x_t_hbm, ids_t_hbm, zf_hbm, zc_hbm, ones_hbm,
          sum_s_hbm, cnt_s_hbm, sum_t_hbm, cnt_t_hbm,
          idx_v, data_v, ones_v, acc_sh, cnt_sh):
        c = lax.axis_index("c")
        s = lax.axis_index("s")
        pltpu.sync_copy(zf_hbm, acc_sh.at[pl.ds(s * ZROWS, ZROWS)])
        pltpu.sync_copy(zc_hbm, cnt_sh.at[pl.ds(s * ZROWS, ZROWS)])
        pltpu.sync_copy(ones_hbm, ones_v)
        plsc.subcore_barrier()

        def side(x_hbm, ids_hbm, sum_hbm, cnt_hbm):
            pltpu.sync_copy(ids_hbm.at[s], idx_v)
            base = s * PER_TILE

            def body(ci, carry):
                pltpu.sync_copy(x_hbm.at[pl.ds(base + ci * CHUNK, CHUNK)],
                                data_v)
                pltpu.sync_copy(data_v, acc_sh.at[idx_v.at[ci]], add=True)
                pltpu.sync_copy(ones_v, cnt_sh.at[idx_v.at[ci]], add=True)
                return carry

            @pl.when(s < TILES - 1)
            def _():
                lax.fori_loop(0, NCH, body, 0)

            @pl.when(s == TILES - 1)
            def _():
                lax.fori_loop(0, LAST_FULL, body, 0)
                pltpu.sync_copy(
                    x_hbm.at[pl.ds(base + LAST_FULL * CHUNK, LAST_REM)],
                    data_v.at[pl.ds(0, LAST_REM)])
                pltpu.sync_copy(data_v, acc_sh.at[idx_v.at[LAST_FULL]],
                                add=True)
                pltpu.sync_copy(ones_v, cnt_sh.at[idx_v.at[LAST_FULL]],
                                add=True)

            plsc.subcore_barrier()
            pltpu.sync_copy(acc_sh.at[pl.ds(s * ZROWS, ZROWS)],
                            sum_hbm.at[pl.ds(s * ZROWS, ZROWS)])
            pltpu.sync_copy(cnt_sh.at[pl.ds(s * ZROWS, ZROWS)],
                            cnt_hbm.at[pl.ds(s * ZROWS, ZROWS)])

        @pl.when(c == 0)
        def _():
            side(x_s_hbm, ids_s_hbm, sum_s_hbm, cnt_s_hbm)

        @pl.when(c == 1)
        def _():
            side(x_t_hbm, ids_t_hbm, sum_t_hbm, cnt_t_hbm)

    return k(x_s, ids_s, x_t, ids_t, zeros_f, zeros_c, ones_c)


def _tc_mlp(sum_s, cnt_s, sum_t, cnt_t, u, w1t, b1, w2t, b2):
    def body(sum_s_ref, cnt_s_ref, sum_t_ref, cnt_t_ref, u_ref,
             w1_ref, b1_ref, w2_ref, b2_ref, out_ref):
        cs = jnp.maximum(cnt_s_ref[:, 0:1], 1.0)
        ct = jnp.maximum(cnt_t_ref[:, 0:1], 1.0)
        ms = sum_s_ref[...] / cs
        mt = sum_t_ref[...] / ct
        h = (jnp.dot(u_ref[...], w1_ref[0:F],
                     preferred_element_type=jnp.float32)
             + jnp.dot(ms, w1_ref[F:2 * F],
                       preferred_element_type=jnp.float32)
             + jnp.dot(mt, w1_ref[2 * F:3 * F],
                       preferred_element_type=jnp.float32)
             + b1_ref[...])
        h = jnp.where(h >= 0, h, 0.1 * h)
        out_ref[...] = (jnp.dot(h, w2_ref[...],
                                preferred_element_type=jnp.float32)
                        + b2_ref[...])

    return pl.pallas_call(
        body,
        out_shape=jax.ShapeDtypeStruct((B, F), jnp.float32),
    )(sum_s, cnt_s, sum_t, cnt_t, u, w1t, b1, w2t, b2)


def kernel(x_s, x_t, edge_index, edge_attr, u, batch_s, batch_t, W1, b1, W2, b2):
    del edge_index, edge_attr
    ids_s = _prep_ids(batch_s)
    ids_t = _prep_ids(batch_t)
    zeros_f = jnp.zeros((ZROWS, F), jnp.float32)
    zeros_c = jnp.zeros((ZROWS, CNT_W), jnp.float32)
    ones_c = jnp.ones((CHUNK, CNT_W), jnp.float32)
    sum_s, cnt_s, sum_t, cnt_t = _sc_segment_sums(
        x_s, ids_s, x_t, ids_t, zeros_f, zeros_c, ones_c)
    return _tc_mlp(sum_s, cnt_s, sum_t, cnt_t, u,
                   W1.T, b1.reshape(1, F), W2.T, b2.reshape(1, F))


# speedup vs baseline: 5.4480x; 1.0962x over previous
import functools

import jax
import jax.numpy as jnp
from jax import lax
from jax.experimental import pallas as pl
from jax.experimental.pallas import tpu as pltpu
from jax.experimental.pallas import tpu_sc as plsc

N = 100000
F = 128
B = 512
TILES = 16
CHUNK = 128
NCH = 49
PER_TILE = NCH * CHUNK
LAST_ROWS = N - 15 * PER_TILE
LAST_FULL = LAST_ROWS // CHUNK
LAST_REM = LAST_ROWS - LAST_FULL * CHUNK
ACC_ROWS = 528
ZROWS = B // TILES
CNT_W = 128


def _prep_ids(batch):
    ids = batch.astype(jnp.int32)
    pad = jnp.full((TILES * PER_TILE - N,), B, jnp.int32)
    return jnp.concatenate([ids, pad]).reshape(TILES, NCH, CHUNK)


def _sc_segment_sums(x_s, ids_s, x_t, ids_t, zeros_f, ones_c):
    mesh = plsc.VectorSubcoreMesh(core_axis_name="c", subcore_axis_name="s")

    @functools.partial(
        pl.kernel,
        out_type=(
            jax.ShapeDtypeStruct((B, F), jnp.float32),
            jax.ShapeDtypeStruct((B, CNT_W), jnp.float32),
            jax.ShapeDtypeStruct((B, F), jnp.float32),
            jax.ShapeDtypeStruct((B, CNT_W), jnp.float32),
        ),
        mesh=mesh,
        scratch_types=[
            pltpu.VMEM((NCH, CHUNK), jnp.int32),
            pltpu.VMEM((2, CHUNK, F), jnp.float32),
            pltpu.VMEM((CHUNK, CNT_W), jnp.float32),
            pltpu.VMEM_SHARED((ACC_ROWS, F), jnp.float32),
            pltpu.VMEM_SHARED((ACC_ROWS, CNT_W), jnp.float32),
            pltpu.SemaphoreType.DMA,
            pltpu.SemaphoreType.DMA,
        ],
    )
    def k(x_s_hbm, ids_s_hbm, x_t_hbm, ids_t_hbm, zf_hbm, ones_hbm,
          sum_s_hbm, cnt_s_hbm, sum_t_hbm, cnt_t_hbm,
          idx_v, data_v, ones_v, acc_sh, cnt_sh, sem0, sem1):
        c = lax.axis_index("c")
        s = lax.axis_index("s")
        pltpu.sync_copy(zf_hbm, acc_sh.at[pl.ds(s * ZROWS, ZROWS)])
        pltpu.sync_copy(zf_hbm, cnt_sh.at[pl.ds(s * ZROWS, ZROWS)])
        pltpu.sync_copy(ones_hbm, ones_v)
        plsc.subcore_barrier()

        def side(x_hbm, ids_hbm, sum_hbm, cnt_hbm):
            pltpu.sync_copy(ids_hbm.at[s], idx_v)
            base = s * PER_TILE

            def ld(ci, b, sem):
                return pltpu.make_async_copy(
                    x_hbm.at[pl.ds(base + ci * CHUNK, CHUNK)],
                    data_v.at[b], sem)

            ld(0, 0, sem0).start()
            ld(1, 1, sem1).start()

            def mk_body(nfull):
                def body(ci, carry):
                    b = ci & 1

                    @pl.when(b == 0)
                    def _():
                        ld(ci, 0, sem0).wait()

                    @pl.when(b == 1)
                    def _():
                        ld(ci, 1, sem1).wait()

                    pltpu.sync_copy(data_v.at[b], acc_sh.at[idx_v.at[ci]],
                                    add=True)
                    pltpu.sync_copy(ones_v, cnt_sh.at[idx_v.at[ci]],
                                    add=True)

                    @pl.when(jnp.logical_and(ci + 2 < nfull, b == 0))
                    def _():
                        ld(ci + 2, 0, sem0).start()

                    @pl.when(jnp.logical_and(ci + 2 < nfull, b == 1))
                    def _():
                        ld(ci + 2, 1, sem1).start()

                    return carry
                return body

            @pl.when(s < TILES - 1)
            def _():
                lax.fori_loop(0, NCH, mk_body(NCH), 0)

            @pl.when(s == TILES - 1)
            def _():
                lax.fori_loop(0, LAST_FULL, mk_body(LAST_FULL), 0)
                pltpu.sync_copy(
                    x_hbm.at[pl.ds(base + LAST_FULL * CHUNK, LAST_REM)],
                    data_v.at[0, pl.ds(0, LAST_REM)])
                pltpu.sync_copy(data_v.at[0], acc_sh.at[idx_v.at[LAST_FULL]],
                                add=True)
                pltpu.sync_copy(ones_v, cnt_sh.at[idx_v.at[LAST_FULL]],
                                add=True)

            plsc.subcore_barrier()
            pltpu.sync_copy(acc_sh.at[pl.ds(s * ZROWS, ZROWS)],
                            sum_hbm.at[pl.ds(s * ZROWS, ZROWS)])
            pltpu.sync_copy(cnt_sh.at[pl.ds(s * ZROWS, ZROWS)],
                            cnt_hbm.at[pl.ds(s * ZROWS, ZROWS)])

        @pl.when(c == 0)
        def _():
            side(x_s_hbm, ids_s_hbm, sum_s_hbm, cnt_s_hbm)

        @pl.when(c == 1)
        def _():
            side(x_t_hbm, ids_t_hbm, sum_t_hbm, cnt_t_hbm)

    return k(x_s, ids_s, x_t, ids_t, zeros_f, ones_c)


def _tc_mlp(sum_s, cnt_s, sum_t, cnt_t, u, w1t, b1, w2t, b2):
    def body(sum_s_ref, cnt_s_ref, sum_t_ref, cnt_t_ref, u_ref,
             w1_ref, b1_ref, w2_ref, b2_ref, out_ref):
        cs = jnp.maximum(cnt_s_ref[:, 0:1], 1.0)
        ct = jnp.maximum(cnt_t_ref[:, 0:1], 1.0)
        ms = sum_s_ref[...] / cs
        mt = sum_t_ref[...] / ct
        h = (jnp.dot(u_ref[...], w1_ref[0:F],
                     preferred_element_type=jnp.float32)
             + jnp.dot(ms, w1_ref[F:2 * F],
                       preferred_element_type=jnp.float32)
             + jnp.dot(mt, w1_ref[2 * F:3 * F],
                       preferred_element_type=jnp.float32)
             + b1_ref[...])
        h = jnp.where(h >= 0, h, 0.1 * h)
        out_ref[...] = (jnp.dot(h, w2_ref[...],
                                preferred_element_type=jnp.float32)
                        + b2_ref[...])

    return pl.pallas_call(
        body,
        out_shape=jax.ShapeDtypeStruct((B, F), jnp.float32),
    )(sum_s, cnt_s, sum_t, cnt_t, u, w1t, b1, w2t, b2)


def kernel(x_s, x_t, edge_index, edge_attr, u, batch_s, batch_t, W1, b1, W2, b2):
    del edge_index, edge_attr
    ids_s = _prep_ids(batch_s)
    ids_t = _prep_ids(batch_t)
    zeros_f = jnp.zeros((ZROWS, F), jnp.float32)
    ones_c = jnp.ones((CHUNK, CNT_W), jnp.float32)
    sum_s, cnt_s, sum_t, cnt_t = _sc_segment_sums(
        x_s, ids_s, x_t, ids_t, zeros_f, ones_c)
    return _tc_mlp(sum_s, cnt_s, sum_t, cnt_t, u,
                   W1.T, b1.reshape(1, F), W2.T, b2.reshape(1, F))


# speedup vs baseline: 8.4348x; 1.5483x over previous
import functools

import jax
import jax.numpy as jnp
from jax import lax
from jax.experimental import pallas as pl
from jax.experimental.pallas import tpu as pltpu
from jax.experimental.pallas import tpu_sc as plsc

N = 100000
F = 128
B = 512
TILES = 16
CHUNK = 128
NCH = 49
PER_TILE = NCH * CHUNK
LAST_ROWS = N - 15 * PER_TILE
LAST_FULL = LAST_ROWS // CHUNK
LAST_REM = LAST_ROWS - LAST_FULL * CHUNK
ACC_ROWS = 528
ZROWS = B // TILES


def _prep_ids(batch):
    ids = batch.astype(jnp.int32)
    pad = jnp.full((TILES * PER_TILE - N,), B, jnp.int32)
    return jnp.concatenate([ids, pad]).reshape(TILES, NCH, CHUNK)


def _sc_segment_sums(x_s, ids_s, x_t, ids_t, zeros_f, ones_1):
    mesh = plsc.VectorSubcoreMesh(core_axis_name="c", subcore_axis_name="s")

    @functools.partial(
        pl.kernel,
        out_type=(
            jax.ShapeDtypeStruct((B, F), jnp.float32),
            jax.ShapeDtypeStruct((B,), jnp.float32),
            jax.ShapeDtypeStruct((B, F), jnp.float32),
            jax.ShapeDtypeStruct((B,), jnp.float32),
        ),
        mesh=mesh,
        scratch_types=[
            pltpu.VMEM((NCH, CHUNK), jnp.int32),
            pltpu.VMEM((2, CHUNK, F), jnp.float32),
            pltpu.VMEM((CHUNK,), jnp.float32),
            pltpu.VMEM((ZROWS,), jnp.float32),
            pltpu.VMEM_SHARED((ACC_ROWS, F), jnp.float32),
            pltpu.VMEM_SHARED((ACC_ROWS,), jnp.float32),
            pltpu.SemaphoreType.DMA,
            pltpu.SemaphoreType.DMA,
        ],
    )
    def k(x_s_hbm, ids_s_hbm, x_t_hbm, ids_t_hbm, zf_hbm, ones_hbm,
          sum_s_hbm, cnt_s_hbm, sum_t_hbm, cnt_t_hbm,
          idx_v, data_v, ones_v, c32_v, acc_sh, cnt_sh, sem0, sem1):
        c = lax.axis_index("c")
        s = lax.axis_index("s")
        pltpu.sync_copy(zf_hbm, acc_sh.at[pl.ds(s * ZROWS, ZROWS)])
        c32_v[pl.ds(0, 16)] = jnp.zeros((16,), jnp.float32)
        c32_v[pl.ds(16, 16)] = jnp.zeros((16,), jnp.float32)
        pltpu.sync_copy(c32_v, cnt_sh.at[pl.ds(s * ZROWS, ZROWS)])
        pltpu.sync_copy(ones_hbm, ones_v)
        plsc.subcore_barrier()

        def side(x_hbm, ids_hbm, sum_hbm, cnt_hbm):
            pltpu.sync_copy(ids_hbm.at[s], idx_v)
            base = s * PER_TILE

            def ld(ci, b, sem):
                return pltpu.make_async_copy(
                    x_hbm.at[pl.ds(base + ci * CHUNK, CHUNK)],
                    data_v.at[b], sem)

            ld(0, 0, sem0).start()
            ld(1, 1, sem1).start()

            def mk_body(nfull):
                def body(ci, carry):
                    b = ci & 1

                    @pl.when(b == 0)
                    def _():
                        ld(ci, 0, sem0).wait()

                    @pl.when(b == 1)
                    def _():
                        ld(ci, 1, sem1).wait()

                    pltpu.sync_copy(data_v.at[b], acc_sh.at[idx_v.at[ci]],
                                    add=True)
                    pltpu.sync_copy(ones_v, cnt_sh.at[idx_v.at[ci]],
                                    add=True)

                    @pl.when(jnp.logical_and(ci + 2 < nfull, b == 0))
                    def _():
                        ld(ci + 2, 0, sem0).start()

                    @pl.when(jnp.logical_and(ci + 2 < nfull, b == 1))
                    def _():
                        ld(ci + 2, 1, sem1).start()

                    return carry
                return body

            @pl.when(s < TILES - 1)
            def _():
                lax.fori_loop(0, NCH, mk_body(NCH), 0)

            @pl.when(s == TILES - 1)
            def _():
                lax.fori_loop(0, LAST_FULL, mk_body(LAST_FULL), 0)
                pltpu.sync_copy(
                    x_hbm.at[pl.ds(base + LAST_FULL * CHUNK, LAST_REM)],
                    data_v.at[0, pl.ds(0, LAST_REM)])
                pltpu.sync_copy(data_v.at[0], acc_sh.at[idx_v.at[LAST_FULL]],
                                add=True)
                pltpu.sync_copy(ones_v, cnt_sh.at[idx_v.at[LAST_FULL]],
                                add=True)

            plsc.subcore_barrier()
            pltpu.sync_copy(acc_sh.at[pl.ds(s * ZROWS, ZROWS)],
                            sum_hbm.at[pl.ds(s * ZROWS, ZROWS)])
            pltpu.sync_copy(cnt_sh.at[pl.ds(s * ZROWS, ZROWS)], c32_v)
            pltpu.sync_copy(c32_v, cnt_hbm.at[pl.ds(s * ZROWS, ZROWS)])

        @pl.when(c == 0)
        def _():
            side(x_s_hbm, ids_s_hbm, sum_s_hbm, cnt_s_hbm)

        @pl.when(c == 1)
        def _():
            side(x_t_hbm, ids_t_hbm, sum_t_hbm, cnt_t_hbm)

    return k(x_s, ids_s, x_t, ids_t, zeros_f, ones_1)


def _tc_mlp(sum_s, cnt_s, sum_t, cnt_t, u, w1t, b1, w2t, b2):
    def body(sum_s_ref, cnt_s_ref, sum_t_ref, cnt_t_ref, u_ref,
             w1_ref, b1_ref, w2_ref, b2_ref, out_ref):
        cs = jnp.maximum(cnt_s_ref[...], 1.0)
        ct = jnp.maximum(cnt_t_ref[...], 1.0)
        ms = sum_s_ref[...] / cs
        mt = sum_t_ref[...] / ct
        h = (jnp.dot(u_ref[...], w1_ref[0:F],
                     preferred_element_type=jnp.float32)
             + jnp.dot(ms, w1_ref[F:2 * F],
                       preferred_element_type=jnp.float32)
             + jnp.dot(mt, w1_ref[2 * F:3 * F],
                       preferred_element_type=jnp.float32)
             + b1_ref[...])
        h = jnp.where(h >= 0, h, 0.1 * h)
        out_ref[...] = (jnp.dot(h, w2_ref[...],
                                preferred_element_type=jnp.float32)
                        + b2_ref[...])

    return pl.pallas_call(
        body,
        out_shape=jax.ShapeDtypeStruct((B, F), jnp.float32),
    )(sum_s, cnt_s, sum_t, cnt_t, u, w1t, b1, w2t, b2)


def kernel(x_s, x_t, edge_index, edge_attr, u, batch_s, batch_t, W1, b1, W2, b2):
    del edge_index, edge_attr
    ids_s = _prep_ids(batch_s)
    ids_t = _prep_ids(batch_t)
    zeros_f = jnp.zeros((ZROWS, F), jnp.float32)
    ones_1 = jnp.ones((CHUNK,), jnp.float32)
    sum_s, cnt_s, sum_t, cnt_t = _sc_segment_sums(
        x_s, ids_s, x_t, ids_t, zeros_f, ones_1)
    return _tc_mlp(sum_s, cnt_s.reshape(B, 1), sum_t, cnt_t.reshape(B, 1), u,
                   W1.T, b1.reshape(1, F), W2.T, b2.reshape(1, F))


# speedup vs baseline: 8.6336x; 1.0236x over previous
import functools

import jax
import jax.numpy as jnp
from jax import lax
from jax.experimental import pallas as pl
from jax.experimental.pallas import tpu as pltpu
from jax.experimental.pallas import tpu_sc as plsc

N = 100000
F = 128
B = 512
TILES = 16
CHUNK = 128
NCH = 49
PER_TILE = NCH * CHUNK
LAST_ROWS = N - 15 * PER_TILE
LAST_FULL = LAST_ROWS // CHUNK
LAST_REM = LAST_ROWS - LAST_FULL * CHUNK
ACC_ROWS = 528
ZROWS = B // TILES


def _prep_ids(batch):
    ids = batch.astype(jnp.int32)
    pad = jnp.full((TILES * PER_TILE - N,), B, jnp.int32)
    return jnp.concatenate([ids, pad]).reshape(TILES, NCH, CHUNK)


def _sc_segment_sums(x_s, ids_s, x_t, ids_t, zeros_f, ones_1):
    mesh = plsc.VectorSubcoreMesh(core_axis_name="c", subcore_axis_name="s")

    @functools.partial(
        pl.kernel,
        out_type=(
            jax.ShapeDtypeStruct((B, F), jnp.float32),
            jax.ShapeDtypeStruct((B,), jnp.float32),
            jax.ShapeDtypeStruct((B, F), jnp.float32),
            jax.ShapeDtypeStruct((B,), jnp.float32),
        ),
        mesh=mesh,
        scratch_types=[
            pltpu.VMEM((NCH, CHUNK), jnp.int32),
            pltpu.VMEM((4, CHUNK, F), jnp.float32),
            pltpu.VMEM((CHUNK,), jnp.float32),
            pltpu.VMEM((ZROWS,), jnp.float32),
            pltpu.VMEM_SHARED((ACC_ROWS, F), jnp.float32),
            pltpu.VMEM_SHARED((ACC_ROWS,), jnp.float32),
            pltpu.SemaphoreType.DMA((4,)),
            pltpu.SemaphoreType.DMA((4,)),
            pltpu.SemaphoreType.DMA,
        ],
    )
    def k(x_s_hbm, ids_s_hbm, x_t_hbm, ids_t_hbm, zf_hbm, ones_hbm,
          sum_s_hbm, cnt_s_hbm, sum_t_hbm, cnt_t_hbm,
          idx_v, data_v, ones_v, c32_v, acc_sh, cnt_sh,
          ld_sems, sc_sems, cnt_sem):
        c = lax.axis_index("c")
        s = lax.axis_index("s")
        pltpu.sync_copy(zf_hbm, acc_sh.at[pl.ds(s * ZROWS, ZROWS)])
        c32_v[pl.ds(0, 16)] = jnp.zeros((16,), jnp.float32)
        c32_v[pl.ds(16, 16)] = jnp.zeros((16,), jnp.float32)
        pltpu.sync_copy(c32_v, cnt_sh.at[pl.ds(s * ZROWS, ZROWS)])
        pltpu.sync_copy(ones_hbm, ones_v)
        plsc.subcore_barrier()

        def side(x_hbm, ids_hbm, sum_hbm, cnt_hbm):
            pltpu.sync_copy(ids_hbm.at[s], idx_v)
            base = s * PER_TILE

            def ld(ci, b):
                return pltpu.make_async_copy(
                    x_hbm.at[pl.ds(base + ci * CHUNK, CHUNK)],
                    data_v.at[b], ld_sems.at[b])

            class _Cp:
                def __init__(self, src, dst, sem):
                    self.args = (src, dst, sem)

                def start(self):
                    pltpu.async_copy(*self.args, add=True)

                def wait(self):
                    pltpu.make_async_copy(*self.args).wait()

            def sc_data(ci, b):
                return _Cp(data_v.at[b], acc_sh.at[idx_v.at[ci]],
                           sc_sems.at[b])

            def sc_cnt(ci):
                return _Cp(ones_v, cnt_sh.at[idx_v.at[ci]], cnt_sem)

            ld(0, 0).start()
            ld(1, 1).start()

            def mk_body(nfull):
                def body(ci, carry):
                    b = ci & 3
                    ld(ci, b).wait()
                    sc_data(ci, b).start()
                    sc_cnt(ci).start()
                    nxt = ci + 2
                    b2 = nxt & 3

                    @pl.when(jnp.logical_and(nxt < nfull, ci >= 2))
                    def _():
                        sc_data(ci - 2, b2).wait()
                        sc_cnt(ci - 2).wait()
                        ld(nxt, b2).start()

                    @pl.when(jnp.logical_and(nxt < nfull, ci < 2))
                    def _():
                        ld(nxt, b2).start()

                    return carry
                return body

            def drain(nfull):
                def dbody(j, carry):
                    sc_data(j, j & 3).wait()
                    sc_cnt(j).wait()
                    return carry
                lax.fori_loop(nfull - 4, nfull, dbody, 0)

            @pl.when(s < TILES - 1)
            def _():
                lax.fori_loop(0, NCH, mk_body(NCH), 0)
                drain(NCH)

            @pl.when(s == TILES - 1)
            def _():
                lax.fori_loop(0, LAST_FULL, mk_body(LAST_FULL), 0)
                drain(LAST_FULL)
                pltpu.sync_copy(
                    x_hbm.at[pl.ds(base + LAST_FULL * CHUNK, LAST_REM)],
                    data_v.at[0, pl.ds(0, LAST_REM)])
                pltpu.sync_copy(data_v.at[0], acc_sh.at[idx_v.at[LAST_FULL]],
                                add=True)
                pltpu.sync_copy(ones_v, cnt_sh.at[idx_v.at[LAST_FULL]],
                                add=True)

            plsc.subcore_barrier()
            pltpu.sync_copy(acc_sh.at[pl.ds(s * ZROWS, ZROWS)],
                            sum_hbm.at[pl.ds(s * ZROWS, ZROWS)])
            pltpu.sync_copy(cnt_sh.at[pl.ds(s * ZROWS, ZROWS)], c32_v)
            pltpu.sync_copy(c32_v, cnt_hbm.at[pl.ds(s * ZROWS, ZROWS)])

        @pl.when(c == 0)
        def _():
            side(x_s_hbm, ids_s_hbm, sum_s_hbm, cnt_s_hbm)

        @pl.when(c == 1)
        def _():
            side(x_t_hbm, ids_t_hbm, sum_t_hbm, cnt_t_hbm)

    return k(x_s, ids_s, x_t, ids_t, zeros_f, ones_1)


def _tc_mlp(sum_s, cnt_s, sum_t, cnt_t, u, w1t, b1, w2t, b2):
    def body(sum_s_ref, cnt_s_ref, sum_t_ref, cnt_t_ref, u_ref,
             w1_ref, b1_ref, w2_ref, b2_ref, out_ref):
        cs = jnp.maximum(cnt_s_ref[...], 1.0)
        ct = jnp.maximum(cnt_t_ref[...], 1.0)
        ms = sum_s_ref[...] / cs
        mt = sum_t_ref[...] / ct
        h = (jnp.dot(u_ref[...], w1_ref[0:F],
                     preferred_element_type=jnp.float32)
             + jnp.dot(ms, w1_ref[F:2 * F],
                       preferred_element_type=jnp.float32)
             + jnp.dot(mt, w1_ref[2 * F:3 * F],
                       preferred_element_type=jnp.float32)
             + b1_ref[...])
        h = jnp.where(h >= 0, h, 0.1 * h)
        out_ref[...] = (jnp.dot(h, w2_ref[...],
                                preferred_element_type=jnp.float32)
                        + b2_ref[...])

    return pl.pallas_call(
        body,
        out_shape=jax.ShapeDtypeStruct((B, F), jnp.float32),
    )(sum_s, cnt_s, sum_t, cnt_t, u, w1t, b1, w2t, b2)


def kernel(x_s, x_t, edge_index, edge_attr, u, batch_s, batch_t, W1, b1, W2, b2):
    del edge_index, edge_attr
    ids_s = _prep_ids(batch_s)
    ids_t = _prep_ids(batch_t)
    zeros_f = jnp.zeros((ZROWS, F), jnp.float32)
    ones_1 = jnp.ones((CHUNK,), jnp.float32)
    sum_s, cnt_s, sum_t, cnt_t = _sc_segment_sums(
        x_s, ids_s, x_t, ids_t, zeros_f, ones_1)
    return _tc_mlp(sum_s, cnt_s.reshape(B, 1), sum_t, cnt_t.reshape(B, 1), u,
                   W1.T, b1.reshape(1, F), W2.T, b2.reshape(1, F))
